# baseline (speedup 1.0000x reference)
import functools

import jax
import jax.numpy as jnp
from jax import lax
from jax.experimental import pallas as pl
from jax.experimental.pallas import tpu as pltpu
from jax.experimental.pallas import tpu_sc as plsc

Q = 784
D = 128
K_TOTAL = 100000
KB = 5000
NB = K_TOTAL // KB
PATCHES_PER_IMAGE = 196
NUM_IMAGES = 4
L = 16
NG = Q // L


def _knn_block_kernel(q_ref, k_ref, mrow_ref, arow_ref):
    i = pl.program_id(0)
    kb = k_ref[...]
    q = q_ref[...]
    h = 0.5 * jnp.sum(kb * kb, axis=1, keepdims=True)
    kq = jax.lax.dot_general(
        kb, q,
        dimension_numbers=(((1,), (1,)), ((), ())),
        preferred_element_type=jnp.float32,
    )
    p = h - kq
    mrow_ref[0] = jnp.min(p, axis=0, keepdims=True)
    arow_ref[0] = jnp.argmin(p, axis=0)[None, :] + i * KB


def _merge_kernel(m_hbm, a_hbm, q2_hbm, d2_hbm, idx_hbm, img_hbm,
                  mv_v, av_v, q2_v, d2s_v, ai_v, img_v):
    wid = lax.axis_index("s") * 2 + lax.axis_index("c")

    @pl.when(wid == 0)
    def _():
        pltpu.sync_copy(m_hbm, mv_v)
        pltpu.sync_copy(a_hbm, av_v)
        pltpu.sync_copy(q2_hbm, q2_v)

        def group(g, _):
            off = g * L
            mv = mv_v[pl.ds(off, L)]
            av = av_v[pl.ds(off, L)]
            for j in range(1, NB):
                cur = mv_v[pl.ds(j * Q + off, L)]
                ci = av_v[pl.ds(j * Q + off, L)]
                better = cur < mv
                mv = jnp.where(better, cur, mv)
                av = jnp.where(better, ci, av)
            d2 = q2_v[pl.ds(off, L)] + 2.0 * mv
            d2s_v[pl.ds(off, L)] = jnp.maximum(d2, 1e-12)
            ai_v[pl.ds(off, L)] = av
            return 0

        lax.fori_loop(0, NG, group, 0)
        d2s_v[pl.ds(Q, L)] = jnp.full((L,), -3e38, jnp.float32)

        lanes = lax.iota(jnp.int32, L)
        for im in range(NUM_IMAGES):
            base = im * PATCHES_PER_IMAGE
            pmax = jnp.full((L,), -3e38, jnp.float32)
            for t in range(PATCHES_PER_IMAGE // L):
                pmax = jnp.maximum(pmax, d2s_v[pl.ds(base + t * L, L)])
            tail = d2s_v[pl.ds(base + (PATCHES_PER_IMAGE // L) * L, L)]
            rem = PATCHES_PER_IMAGE % L
            pmax = jnp.maximum(pmax, jnp.where(lanes < rem, tail, -3e38))
            img_v[pl.ds(im * L, L)] = pmax

        pltpu.sync_copy(d2s_v.at[pl.ds(0, Q)], d2_hbm)
        pltpu.sync_copy(ai_v.at[pl.ds(0, Q)], idx_hbm)
        pltpu.sync_copy(img_v, img_hbm)


def _sqrt_kernel(d2_ref, imgd2_ref, patch_ref, img_ref):
    patch_ref[...] = jnp.sqrt(d2_ref[...])
    img_ref[...] = jnp.sqrt(jnp.max(imgd2_ref[...], axis=1, keepdims=True))


@jax.jit
def _run(queries, keys):
    q2 = jnp.sum(queries * queries, axis=1)
    mrow, arow = pl.pallas_call(
        _knn_block_kernel,
        grid=(NB,),
        in_specs=[
            pl.BlockSpec((Q, D), lambda i: (0, 0)),
            pl.BlockSpec((KB, D), lambda i: (i, 0)),
        ],
        out_specs=[
            pl.BlockSpec((1, 1, Q), lambda i: (i, 0, 0)),
            pl.BlockSpec((1, 1, Q), lambda i: (i, 0, 0)),
        ],
        out_shape=[
            jax.ShapeDtypeStruct((NB, 1, Q), jnp.float32),
            jax.ShapeDtypeStruct((NB, 1, Q), jnp.int32),
        ],
        compiler_params=pltpu.CompilerParams(
            dimension_semantics=("arbitrary",),
        ),
    )(queries, keys)

    mesh = plsc.VectorSubcoreMesh(core_axis_name="c", subcore_axis_name="s")
    merge = functools.partial(
        pl.kernel, mesh=mesh,
        out_type=[
            jax.ShapeDtypeStruct((Q,), jnp.float32),
            jax.ShapeDtypeStruct((Q,), jnp.int32),
            jax.ShapeDtypeStruct((NUM_IMAGES * L,), jnp.float32),
        ],
        scratch_types=[
            pltpu.VMEM((NB * Q,), jnp.float32),
            pltpu.VMEM((NB * Q,), jnp.int32),
            pltpu.VMEM((Q,), jnp.float32),
            pltpu.VMEM((Q + L,), jnp.float32),
            pltpu.VMEM((Q,), jnp.int32),
            pltpu.VMEM((NUM_IMAGES * L,), jnp.float32),
        ],
    )(_merge_kernel)
    d2min, idx, imgd2 = merge(mrow.reshape(NB * Q), arow.reshape(NB * Q), q2)

    patch, img = pl.pallas_call(
        _sqrt_kernel,
        out_shape=[
            jax.ShapeDtypeStruct((1, Q), jnp.float32),
            jax.ShapeDtypeStruct((NUM_IMAGES, 1), jnp.float32),
        ],
    )(d2min.reshape(1, Q), imgd2.reshape(NUM_IMAGES, L))
    return patch[0], idx, img[:, 0]


def kernel(queries, keys, batchsize):
    patch, idx, img = _run(queries, keys)
    batch_dep = (0 * jnp.asarray(batchsize)).astype(patch.dtype)
    return img + batch_dep, patch, idx


# speedup vs baseline: 1.3740x; 1.3740x over previous
import jax
import jax.numpy as jnp
from jax.experimental import pallas as pl
from jax.experimental.pallas import tpu as pltpu

Q = 784
D = 128
K_TOTAL = 100000
KB = 5000
NB = K_TOTAL // KB
PATCHES_PER_IMAGE = 196
NUM_IMAGES = 4


def _knn_kernel(q_ref, k_ref, patch_ref, idx_ref, img_ref, mval, midx):
    i = pl.program_id(0)

    kb = k_ref[...]
    q = q_ref[...]
    h = 0.5 * jnp.sum(kb * kb, axis=1, keepdims=True)
    kq = jax.lax.dot_general(
        kb, q,
        dimension_numbers=(((1,), (1,)), ((), ())),
        preferred_element_type=jnp.float32,
    )
    p = h - kq
    m = jnp.min(p, axis=0, keepdims=True)
    a = jnp.argmin(p, axis=0)[None, :] + i * KB

    @pl.when(i == 0)
    def _():
        mval[...] = m
        midx[...] = a

    @pl.when(i > 0)
    def _():
        better = m < mval[...]
        mval[...] = jnp.where(better, m, mval[...])
        midx[...] = jnp.where(better, a, midx[...])

    @pl.when(i == NB - 1)
    def _():
        ones = jnp.ones((1, D), jnp.float32)
        q2 = jax.lax.dot_general(
            ones, q * q,
            dimension_numbers=(((1,), (1,)), ((), ())),
            preferred_element_type=jnp.float32,
        )
        d2 = q2 + 2.0 * mval[...]
        ps = jnp.sqrt(jnp.maximum(d2, 1e-12))
        patch_ref[...] = ps
        idx_ref[...] = midx[...]
        for j in range(NUM_IMAGES):
            chunk = ps[:, j * PATCHES_PER_IMAGE:(j + 1) * PATCHES_PER_IMAGE]
            img_ref[:, j:j + 1] = jnp.max(chunk, axis=1, keepdims=True)


@jax.jit
def _run(queries, keys):
    patch, idx, img = pl.pallas_call(
        _knn_kernel,
        grid=(NB,),
        in_specs=[
            pl.BlockSpec((Q, D), lambda i: (0, 0)),
            pl.BlockSpec((KB, D), lambda i: (i, 0)),
        ],
        out_specs=[
            pl.BlockSpec((1, Q), lambda i: (0, 0)),
            pl.BlockSpec((1, Q), lambda i: (0, 0)),
            pl.BlockSpec((1, NUM_IMAGES), lambda i: (0, 0)),
        ],
        out_shape=[
            jax.ShapeDtypeStruct((1, Q), jnp.float32),
            jax.ShapeDtypeStruct((1, Q), jnp.int32),
            jax.ShapeDtypeStruct((1, NUM_IMAGES), jnp.float32),
        ],
        scratch_shapes=[
            pltpu.VMEM((1, Q), jnp.float32),
            pltpu.VMEM((1, Q), jnp.int32),
        ],
        compiler_params=pltpu.CompilerParams(
            dimension_semantics=("arbitrary",),
        ),
    )(queries, keys)
    return patch, idx, img


def kernel(queries, keys, batchsize):
    patch, idx, img = _run(queries, keys)
    batch_dep = (0 * jnp.asarray(batchsize)).astype(patch.dtype)
    image_scores = img[0] + batch_dep
    return image_scores, patch[0], idx[0]


# speedup vs baseline: 1.4265x; 1.0382x over previous
import jax
import jax.numpy as jnp
from jax.experimental import pallas as pl
from jax.experimental.pallas import tpu as pltpu

Q = 784
D = 128
K_TOTAL = 100000
KB = 5000
NB = K_TOTAL // KB
PATCHES_PER_IMAGE = 196
NUM_IMAGES = 4


def _knn_kernel(q_ref, k_ref, patch_ref, idx_ref, img_ref, mval, midx):
    i = pl.program_id(0)

    kb = k_ref[...]
    q = q_ref[...]
    h = 0.5 * jnp.sum(kb * kb, axis=1, keepdims=True)
    kq = jax.lax.dot_general(
        kb, q,
        dimension_numbers=(((1,), (1,)), ((), ())),
        preferred_element_type=jnp.float32,
    )
    p = h - kq
    m = jnp.min(p, axis=0, keepdims=True)
    a = jnp.argmin(p, axis=0)[None, :] + i * KB

    @pl.when(i == 0)
    def _():
        mval[...] = m
        midx[...] = a

    @pl.when(i > 0)
    def _():
        better = m < mval[...]
        mval[...] = jnp.where(better, m, mval[...])
        midx[...] = jnp.where(better, a, midx[...])

    @pl.when(i == NB - 1)
    def _():
        ones = jnp.ones((1, D), jnp.float32)
        q2 = jax.lax.dot_general(
            ones, q * q,
            dimension_numbers=(((1,), (1,)), ((), ())),
            preferred_element_type=jnp.float32,
        )
        d2 = q2 + 2.0 * mval[...]
        ps = jnp.sqrt(jnp.maximum(d2, 1e-12))
        patch_ref[...] = ps[0]
        idx_ref[...] = midx[...][0]
        for j in range(NUM_IMAGES):
            chunk = ps[:, j * PATCHES_PER_IMAGE:(j + 1) * PATCHES_PER_IMAGE]
            img_ref[j:j + 1] = jnp.max(chunk, axis=1, keepdims=True)[0]


@jax.jit
def _run(queries, keys):
    patch, idx, img = pl.pallas_call(
        _knn_kernel,
        grid=(NB,),
        in_specs=[
            pl.BlockSpec((Q, D), lambda i: (0, 0)),
            pl.BlockSpec((KB, D), lambda i: (i, 0)),
        ],
        out_specs=[
            pl.BlockSpec((Q,), lambda i: (0,)),
            pl.BlockSpec((Q,), lambda i: (0,)),
            pl.BlockSpec((NUM_IMAGES,), lambda i: (0,)),
        ],
        out_shape=[
            jax.ShapeDtypeStruct((Q,), jnp.float32),
            jax.ShapeDtypeStruct((Q,), jnp.int32),
            jax.ShapeDtypeStruct((NUM_IMAGES,), jnp.float32),
        ],
        scratch_shapes=[
            pltpu.VMEM((1, Q), jnp.float32),
            pltpu.VMEM((1, Q), jnp.int32),
        ],
        compiler_params=pltpu.CompilerParams(
            dimension_semantics=("arbitrary",),
        ),
    )(queries, keys)
    return patch, idx, img


def kernel(queries, keys, batchsize):
    patch, idx, img = _run(queries, keys)
    return img, patch, idx
